# baseline (speedup 1.0000x reference)
import jax
import jax.numpy as jnp
from jax import lax
from jax.experimental import pallas as pl
from jax.experimental.pallas import tpu as pltpu
from jax.experimental.pallas import tpu_sc as plsc

N = 10000
E = 320000
D = 128
NC = 2
NS = 16
NW = NC * NS
CB = 128
NCHT = E // CB
CHW = NCHT // NW
CXT = NCHT - CHW * NW
NP = 10240
RPT = NP // NS
NPA = 10000
RPA = 632
RPL = NPA - 15 * RPA
DEGQ = 6
AGQ = 3

_MESH = plsc.VectorSubcoreMesh(core_axis_name="c", subcore_axis_name="s")


def _wid_base_cnt():
    c = lax.axis_index("c")
    s = lax.axis_index("s")
    wid = s * NC + c
    base = CHW * wid + jnp.minimum(wid, CXT)
    cnt = CHW + jnp.where(wid < CXT, 1, 0)
    return c, s, wid, base, cnt


def _deg_body(ei3, ones, zeros, deg_out, hist, dbuf, ones_v, semi):
    c, s, wid, base, cnt = _wid_base_cnt()
    pltpu.sync_copy(ones, ones_v)
    pltpu.sync_copy(zeros, hist.at[pl.ds(s * RPT, RPT)])
    plsc.subcore_barrier()

    for b in range(DEGQ):
        pltpu.async_copy(ei3.at[1].at[base + b], dbuf.at[b], semi)

    def body(g, carry):
        for b in range(DEGQ):
            k = g * DEGQ + b
            pltpu.make_async_copy(ei3.at[1].at[base + k], dbuf.at[b],
                                  semi).wait()
            pltpu.sync_copy(ones_v, hist.at[dbuf.at[b]], add=True)

            @pl.when(k + DEGQ < cnt)
            def _():
                pltpu.async_copy(ei3.at[1].at[base + k + DEGQ], dbuf.at[b],
                                 semi)
        return carry

    lax.fori_loop(0, CHW // DEGQ, body, 0)

    @pl.when(wid < CXT)
    def _():
        b = CHW % DEGQ
        pltpu.make_async_copy(ei3.at[1].at[base + CHW], dbuf.at[b],
                              semi).wait()
        pltpu.sync_copy(ones_v, hist.at[dbuf.at[b]], add=True)

    plsc.subcore_barrier()
    pltpu.sync_copy(hist.at[pl.ds(s * RPT, RPT)],
                    deg_out.at[c].at[pl.ds(s * RPT, RPT)])


_deg_call = pl.kernel(
    _deg_body,
    out_type=jax.ShapeDtypeStruct((NC, NP), jnp.float32),
    mesh=_MESH,
    scratch_types=[
        pltpu.VMEM_SHARED((NP,), jnp.float32),
        pltpu.VMEM((DEGQ, CB), jnp.int32),
        pltpu.VMEM((CB,), jnp.float32),
        pltpu.SemaphoreType.DMA,
    ],
)


IQ = 2 * AGQ


def _agg_body(ei3, xp, zeros, agg_out, acc, sbuf, dbuf, rows,
              semi0, semi1, semi2, semi3, semi4, semi5,
              semg0, semg1, semg2, semsc0, semsc1, semsc2):
    c, s, wid, base, cnt = _wid_base_cnt()
    semi = (semi0, semi1, semi2, semi3, semi4, semi5)
    semg = (semg0, semg1, semg2)
    semsc = (semsc0, semsc1, semsc2)
    @pl.when(s < 15)
    def _():
        pltpu.sync_copy(zeros, acc.at[pl.ds(s * RPA, RPA)])

    @pl.when(s == 15)
    def _():
        pltpu.sync_copy(zeros.at[pl.ds(0, RPL)], acc.at[pl.ds(15 * RPA, RPL)])

    plsc.subcore_barrier()

    def load_idx(k, bs, bd):
        pltpu.async_copy(ei3.at[0].at[base + k], sbuf.at[bs], semi[bd])
        pltpu.async_copy(ei3.at[1].at[base + k], dbuf.at[bd], semi[bd])

    def wait_idx(k, bs, bd):
        pltpu.make_async_copy(ei3.at[0].at[base + k], sbuf.at[bs],
                              semi[bd]).wait()
        pltpu.make_async_copy(ei3.at[1].at[base + k], dbuf.at[bd],
                              semi[bd]).wait()

    def fire_gather(bs, b):
        pltpu.async_copy(xp.at[sbuf.at[bs]], rows.at[b], semg[b])

    def wait_gather(bs, b):
        pltpu.make_async_copy(xp.at[sbuf.at[bs]], rows.at[b],
                              semg[b]).wait()

    def fire_scatter(bd, b):
        pltpu.async_copy(rows.at[b], acc.at[dbuf.at[bd]], semsc[b],
                         add=True)

    def wait_scatter(bd, b):
        pltpu.make_async_copy(rows.at[b], acc.at[dbuf.at[bd]],
                              semsc[b]).wait()

    load_idx(0, 0, 0)
    load_idx(1, 1, 1)
    wait_idx(0, 0, 0)
    fire_gather(0, 0)

    def body(g, carry):
        for u in range(IQ):
            k = g * IQ + u
            b = u % AGQ
            b1 = (u + 1) % AGQ

            @pl.when(k + 1 < cnt)
            def _():
                wait_idx(k + 1, b1, (u + 1) % IQ)

                @pl.when(k >= 2)
                def _():
                    wait_scatter((u + 4) % IQ, b1)

                fire_gather(b1, b1)

            wait_gather(b, b)
            fire_scatter(u, b)

            @pl.when(k + 2 < cnt)
            def _():
                load_idx(k + 2, (u + 2) % AGQ, (u + 2) % IQ)
        return carry

    lax.fori_loop(0, CHW // IQ, body, 0)

    @pl.when(wid < CXT)
    def _():
        wait_gather(CHW % AGQ, CHW % AGQ)
        fire_scatter(CHW % IQ, CHW % AGQ)

    @pl.when(wid >= CXT)
    def _():
        wait_scatter((CHW - 3) % IQ, (CHW - 3) % AGQ)

    wait_scatter((CHW - 2) % IQ, (CHW - 2) % AGQ)
    wait_scatter((CHW - 1) % IQ, (CHW - 1) % AGQ)

    @pl.when(wid < CXT)
    def _():
        wait_scatter(CHW % IQ, CHW % AGQ)

    plsc.subcore_barrier()

    @pl.when(s < 15)
    def _():
        pltpu.sync_copy(acc.at[pl.ds(s * RPA, RPA)],
                        agg_out.at[c].at[pl.ds(s * RPA, RPA)])

    @pl.when(s == 15)
    def _():
        pltpu.sync_copy(acc.at[pl.ds(15 * RPA, RPL)],
                        agg_out.at[c].at[pl.ds(15 * RPA, RPL)])


_agg_call = pl.kernel(
    _agg_body,
    out_type=jax.ShapeDtypeStruct((NC, NPA, D), jnp.float32),
    mesh=_MESH,
    scratch_types=[
        pltpu.VMEM_SHARED((NPA, D), jnp.float32),
        pltpu.VMEM((AGQ, CB), jnp.int32),
        pltpu.VMEM((IQ, CB), jnp.int32),
        pltpu.VMEM((AGQ, CB, D), jnp.float32),
        pltpu.SemaphoreType.DMA,
        pltpu.SemaphoreType.DMA,
        pltpu.SemaphoreType.DMA,
        pltpu.SemaphoreType.DMA,
        pltpu.SemaphoreType.DMA,
        pltpu.SemaphoreType.DMA,
        pltpu.SemaphoreType.DMA,
        pltpu.SemaphoreType.DMA,
        pltpu.SemaphoreType.DMA,
        pltpu.SemaphoreType.DMA,
        pltpu.SemaphoreType.DMA,
        pltpu.SemaphoreType.DMA,
    ],
)


def _prep_body(deg_ref, x_ref, xp_ref, norm_ref):
    d = (deg_ref[0, :N] + deg_ref[1, :N])[:, None]
    norm = lax.rsqrt(jnp.maximum(d, 1.0))
    norm_ref[...] = norm
    xp_ref[...] = x_ref[...] * norm


_prep_call = pl.pallas_call(
    _prep_body,
    out_shape=(
        jax.ShapeDtypeStruct((N, D), jnp.float32),
        jax.ShapeDtypeStruct((N, 1), jnp.float32),
    ),
)


def _fin_body(agg_ref, norm_ref, w_ref, b_ref, o_ref):
    a = (agg_ref[0, :N] + agg_ref[1, :N]) * norm_ref[...]
    acc = jnp.dot(a, w_ref[...], preferred_element_type=jnp.float32)
    o_ref[...] = jnp.maximum(acc + b_ref[...], 0.0)


_fin_call = pl.pallas_call(
    _fin_body,
    out_shape=jax.ShapeDtypeStruct((N, D), jnp.float32),
)


def kernel(t, x, edge_index, W, b):
    ei3 = edge_index.reshape(2, NCHT, CB)
    ones = jnp.ones((CB,), jnp.float32)
    zeros_h = jnp.zeros((RPT,), jnp.float32)
    zeros_a = jnp.zeros((RPA, D), jnp.float32)
    deg = _deg_call(ei3, ones, zeros_h)
    xp, norm = _prep_call(deg, x)
    agg2 = _agg_call(ei3, xp, zeros_a)
    return _fin_call(agg2, norm, W, b.reshape(1, D))


# speedup vs baseline: 1.0167x; 1.0167x over previous
import jax
import jax.numpy as jnp
from jax import lax
from jax.experimental import pallas as pl
from jax.experimental.pallas import tpu as pltpu
from jax.experimental.pallas import tpu_sc as plsc

N = 10000
E = 320000
D = 128
NC = 2
NS = 16
NW = NC * NS
CB = 128
NCHT = E // CB
CHW = NCHT // NW
CXT = NCHT - CHW * NW
NP = 10240
RPT = NP // NS
NPA = 10112
RPA = NPA // NS
DEGQ = 6
AGQ = 3

_MESH = plsc.VectorSubcoreMesh(core_axis_name="c", subcore_axis_name="s")


def _wid_base_cnt():
    c = lax.axis_index("c")
    s = lax.axis_index("s")
    wid = s * NC + c
    base = CHW * wid + jnp.minimum(wid, CXT)
    cnt = CHW + jnp.where(wid < CXT, 1, 0)
    return c, s, wid, base, cnt


def _deg_body(ei3, ones, zeros, deg_out, hist, dbuf, ones_v, semi):
    c, s, wid, base, cnt = _wid_base_cnt()
    pltpu.sync_copy(ones, ones_v)
    pltpu.sync_copy(zeros, hist.at[pl.ds(s * RPT, RPT)])
    plsc.subcore_barrier()

    for b in range(DEGQ):
        pltpu.async_copy(ei3.at[1].at[base + b], dbuf.at[b], semi)

    def body(g, carry):
        for b in range(DEGQ):
            k = g * DEGQ + b
            pltpu.make_async_copy(ei3.at[1].at[base + k], dbuf.at[b],
                                  semi).wait()
            pltpu.sync_copy(ones_v, hist.at[dbuf.at[b]], add=True)

            @pl.when(k + DEGQ < cnt)
            def _():
                pltpu.async_copy(ei3.at[1].at[base + k + DEGQ], dbuf.at[b],
                                 semi)
        return carry

    lax.fori_loop(0, CHW // DEGQ, body, 0)

    @pl.when(wid < CXT)
    def _():
        b = CHW % DEGQ
        pltpu.make_async_copy(ei3.at[1].at[base + CHW], dbuf.at[b],
                              semi).wait()
        pltpu.sync_copy(ones_v, hist.at[dbuf.at[b]], add=True)

    plsc.subcore_barrier()
    pltpu.sync_copy(hist.at[pl.ds(s * RPT, RPT)],
                    deg_out.at[c].at[pl.ds(s * RPT, RPT)])


_deg_call = pl.kernel(
    _deg_body,
    out_type=jax.ShapeDtypeStruct((NC, NP), jnp.float32),
    mesh=_MESH,
    scratch_types=[
        pltpu.VMEM_SHARED((NP,), jnp.float32),
        pltpu.VMEM((DEGQ, CB), jnp.int32),
        pltpu.VMEM((CB,), jnp.float32),
        pltpu.SemaphoreType.DMA,
    ],
)


def _agg_body(ei3, xp, zeros, agg_out, acc, sbuf, dbuf, rows,
              semi0, semi1, semi2, semg0, semg1, semg2):
    c, s, wid, base, cnt = _wid_base_cnt()
    semi = (semi0, semi1, semi2)
    semg = (semg0, semg1, semg2)
    pltpu.sync_copy(zeros, acc.at[pl.ds(s * RPA, RPA)])
    plsc.subcore_barrier()

    def load_idx(k, b):
        pltpu.async_copy(ei3.at[0].at[base + k], sbuf.at[b], semi[b])
        pltpu.async_copy(ei3.at[1].at[base + k], dbuf.at[b], semi[b])

    def wait_idx(k, b):
        pltpu.make_async_copy(ei3.at[0].at[base + k], sbuf.at[b],
                              semi[b]).wait()
        pltpu.make_async_copy(ei3.at[1].at[base + k], dbuf.at[b],
                              semi[b]).wait()

    def fire_gather(b):
        pltpu.async_copy(xp.at[sbuf.at[b]], rows.at[b], semg[b])

    def wait_gather(b):
        pltpu.make_async_copy(xp.at[sbuf.at[b]], rows.at[b], semg[b]).wait()

    for b in range(AGQ):
        load_idx(b, b)
    wait_idx(0, 0)
    fire_gather(0)

    def body(g, carry):
        for b in range(AGQ):
            k = g * AGQ + b
            b1 = (b + 1) % AGQ

            @pl.when(k + 1 < cnt)
            def _():
                wait_idx(k + 1, b1)
                fire_gather(b1)

            wait_gather(b)
            pltpu.sync_copy(rows.at[b], acc.at[dbuf.at[b]], add=True)

            @pl.when(k + AGQ < cnt)
            def _():
                load_idx(k + AGQ, b)
        return carry

    lax.fori_loop(0, CHW // AGQ, body, 0)

    @pl.when(wid < CXT)
    def _():
        b = CHW % AGQ
        wait_gather(b)
        pltpu.sync_copy(rows.at[b], acc.at[dbuf.at[b]], add=True)

    plsc.subcore_barrier()
    pltpu.sync_copy(acc.at[pl.ds(s * RPA, RPA)],
                    agg_out.at[c].at[pl.ds(s * RPA, RPA)])


_agg_call = pl.kernel(
    _agg_body,
    out_type=jax.ShapeDtypeStruct((NC, NPA, D), jnp.float32),
    mesh=_MESH,
    scratch_types=[
        pltpu.VMEM_SHARED((NPA, D), jnp.float32),
        pltpu.VMEM((AGQ, CB), jnp.int32),
        pltpu.VMEM((AGQ, CB), jnp.int32),
        pltpu.VMEM((AGQ, CB, D), jnp.float32),
        pltpu.SemaphoreType.DMA,
        pltpu.SemaphoreType.DMA,
        pltpu.SemaphoreType.DMA,
        pltpu.SemaphoreType.DMA,
        pltpu.SemaphoreType.DMA,
        pltpu.SemaphoreType.DMA,
    ],
)


def _prep_body(deg_ref, x_ref, xp_ref, norm_ref):
    d = (deg_ref[0, :N] + deg_ref[1, :N])[:, None]
    norm = lax.rsqrt(jnp.maximum(d, 1.0))
    norm_ref[...] = norm
    xp_ref[...] = x_ref[...] * norm


_prep_call = pl.pallas_call(
    _prep_body,
    out_shape=(
        jax.ShapeDtypeStruct((N, D), jnp.float32),
        jax.ShapeDtypeStruct((N, 1), jnp.float32),
    ),
)


def _fin_body(agg_ref, norm_ref, w_ref, b_ref, o_ref):
    a = (agg_ref[0, :N] + agg_ref[1, :N]) * norm_ref[...]
    acc = jnp.dot(a, w_ref[...], preferred_element_type=jnp.float32)
    o_ref[...] = jnp.maximum(acc + b_ref[...], 0.0)


_fin_call = pl.pallas_call(
    _fin_body,
    out_shape=jax.ShapeDtypeStruct((N, D), jnp.float32),
)


def kernel(t, x, edge_index, W, b):
    ei3 = edge_index.reshape(2, NCHT, CB)
    ones = jnp.ones((CB,), jnp.float32)
    zeros_h = jnp.zeros((RPT,), jnp.float32)
    zeros_a = jnp.zeros((RPA, D), jnp.float32)
    deg = _deg_call(ei3, ones, zeros_h)
    xp, norm = _prep_call(deg, x)
    agg2 = _agg_call(ei3, xp, zeros_a)
    return _fin_call(agg2, norm, W, b.reshape(1, D))
